# baseline (speedup 1.0000x reference)
import functools

import jax
import jax.numpy as jnp
from jax import lax
from jax.experimental import pallas as pl
from jax.experimental.pallas import tpu as pltpu
from jax.experimental.pallas import tpu_sc as plsc

EMBED = 64
BATCH = 16384

NC = 2
NS = 16
NW = NC * NS
B_PER_W = BATCH // NW
DCHUNK = 8

_mesh = plsc.VectorSubcoreMesh(core_axis_name="c", subcore_axis_name="s")


@functools.partial(
    pl.kernel,
    mesh=_mesh,
    compiler_params=pltpu.CompilerParams(use_tc_tiling_on_sc=False),
    out_type=[
        jax.ShapeDtypeStruct((EMBED, BATCH), jnp.float32),
        jax.ShapeDtypeStruct((EMBED, BATCH), jnp.float32),
        jax.ShapeDtypeStruct((EMBED, BATCH), jnp.float32),
    ],
    scratch_types=[
        pltpu.VMEM((B_PER_W,), jnp.int32),
        pltpu.VMEM((B_PER_W,), jnp.int32),
        pltpu.VMEM((B_PER_W,), jnp.int32),
        pltpu.VMEM((EMBED, B_PER_W), jnp.float32),
        pltpu.VMEM((EMBED, B_PER_W), jnp.float32),
        pltpu.VMEM((EMBED, B_PER_W), jnp.float32),
        pltpu.SemaphoreType.DMA,
        pltpu.SemaphoreType.DMA,
    ],
)
def _gather3(users_hbm, pos_hbm, neg_hbm, uemb_hbm, iemb_hbm,
             out_u, out_p, out_n,
             idx_u, idx_p, idx_n, rows_u, rows_p, rows_n, sem_g, sem_s):
    wid = lax.axis_index("s") * NC + lax.axis_index("c")
    base = wid * B_PER_W

    pltpu.sync_copy(users_hbm.at[pl.ds(base, B_PER_W)], idx_u)
    pltpu.sync_copy(pos_hbm.at[pl.ds(base, B_PER_W)], idx_p)
    pltpu.sync_copy(neg_hbm.at[pl.ds(base, B_PER_W)], idx_n)

    def step(dc, _):
        copies = []
        for j in range(DCHUNK):
            d = dc * DCHUNK + j
            copies.append(pltpu.async_copy(
                uemb_hbm.at[d].at[idx_u], rows_u.at[d], sem_g))
            copies.append(pltpu.async_copy(
                iemb_hbm.at[d].at[idx_p], rows_p.at[d], sem_g))
            copies.append(pltpu.async_copy(
                iemb_hbm.at[d].at[idx_n], rows_n.at[d], sem_g))
        for c in copies:
            c.wait()

    lax.fori_loop(0, EMBED // DCHUNK, step, None)

    out_sl = pl.ds(base, B_PER_W)
    stores = [
        pltpu.async_copy(rows_u, out_u.at[:, out_sl], sem_s),
        pltpu.async_copy(rows_p, out_p.at[:, out_sl], sem_s),
        pltpu.async_copy(rows_n, out_n.at[:, out_sl], sem_s),
    ]
    for s in stores:
        s.wait()


def kernel(users, pos_items, neg_items, _, user_emb, item_emb):
    u = users.astype(jnp.int32)
    p = pos_items.astype(jnp.int32)
    n = neg_items.astype(jnp.int32)
    out_u, out_p, out_n = _gather3(u, p, n, user_emb.T, item_emb.T)
    return out_u.T, out_p.T, out_n.T, _


# speedup vs baseline: 8.5664x; 8.5664x over previous
import functools

import jax
import jax.numpy as jnp
from jax import lax
from jax.experimental import pallas as pl
from jax.experimental.pallas import tpu as pltpu
from jax.experimental.pallas import tpu_sc as plsc

EMBED = 64
BATCH = 16384
PAIR_ROWS = 500000

NC = 2
NS = 16
NW = NC * NS
B_PER_W = BATCH // NW
VL = 16

_mesh = plsc.VectorSubcoreMesh(core_axis_name="c", subcore_axis_name="s")


@functools.partial(
    pl.kernel,
    mesh=_mesh,
    compiler_params=pltpu.CompilerParams(needs_layout_passes=False),
    out_type=[
        jax.ShapeDtypeStruct((EMBED, BATCH), jnp.float32),
        jax.ShapeDtypeStruct((EMBED, BATCH), jnp.float32),
        jax.ShapeDtypeStruct((EMBED, BATCH), jnp.float32),
    ],
    scratch_types=[
        pltpu.VMEM((B_PER_W,), jnp.int32),
        pltpu.VMEM((B_PER_W,), jnp.int32),
        pltpu.VMEM((B_PER_W, 2 * EMBED), jnp.float32),
        pltpu.VMEM((EMBED, B_PER_W), jnp.float32),
        pltpu.SemaphoreType.DMA,
        pltpu.SemaphoreType.DMA,
    ],
)
def _gather3(users_hbm, pos_hbm, neg_hbm, uemb_hbm, iemb_hbm,
             out_u, out_p, out_n,
             idx_v, idx2, pairs, out_st, sem_g, sem_s):
    wid = lax.axis_index("s") * NC + lax.axis_index("c")
    base = wid * B_PER_W
    out_sl = pl.ds(base, B_PER_W)

    def one_table(idx_hbm, tab_hbm, out_hbm):
        pltpu.sync_copy(idx_hbm.at[pl.ds(base, B_PER_W)], idx_v)

        def shift(m, _):
            idx2[pl.ds(m * VL, VL)] = lax.shift_right_logical(
                idx_v[pl.ds(m * VL, VL)], 1)

        lax.fori_loop(0, B_PER_W // VL, shift, None, unroll=8)

        copies = [
            pltpu.async_copy(
                tab_hbm.at[idx2.at[pl.ds(j * 128, 128)]],
                pairs.at[pl.ds(j * 128, 128), :], sem_g)
            for j in range(B_PER_W // 128)
        ]
        for c in copies:
            c.wait()

        lanes = lax.iota(jnp.int32, VL)

        def extract(m, _):
            i0 = m * VL
            ev = idx_v[pl.ds(i0, VL)]
            cols0 = lax.mul(lax.bitwise_and(ev, 1), EMBED)
            rows = lax.add(lax.broadcast(i0, (VL,)), lanes)
            for c in range(EMBED):
                v = plsc.load_gather(pairs, [rows, lax.add(cols0, c)])
                out_st[c, pl.ds(i0, VL)] = v

        lax.fori_loop(0, B_PER_W // VL, extract, None)

        pltpu.async_copy(out_st, out_hbm.at[:, out_sl], sem_s).wait()

    one_table(users_hbm, uemb_hbm, out_u)
    one_table(pos_hbm, iemb_hbm, out_p)
    one_table(neg_hbm, iemb_hbm, out_n)


def kernel(users, pos_items, neg_items, _, user_emb, item_emb):
    u = users.astype(jnp.int32)
    p = pos_items.astype(jnp.int32)
    n = neg_items.astype(jnp.int32)
    ut = user_emb.reshape(PAIR_ROWS, 2 * EMBED)
    it = item_emb.reshape(PAIR_ROWS, 2 * EMBED)
    out_u, out_p, out_n = _gather3(u, p, n, ut, it)
    return out_u.T, out_p.T, out_n.T, _


# speedup vs baseline: 8.8882x; 1.0376x over previous
import functools

import jax
import jax.numpy as jnp
from jax import lax
from jax.experimental import pallas as pl
from jax.experimental.pallas import tpu as pltpu
from jax.experimental.pallas import tpu_sc as plsc

EMBED = 64
BATCH = 16384

NC = 2
NS = 16
NW = NC * NS
B_PER_W = BATCH // NW
CHUNK = 128
NCHUNK = B_PER_W // CHUNK

_mesh = plsc.VectorSubcoreMesh(core_axis_name="c", subcore_axis_name="s")


@functools.partial(
    pl.kernel,
    mesh=_mesh,
    compiler_params=pltpu.CompilerParams(use_tc_tiling_on_sc=False),
    out_type=[
        jax.ShapeDtypeStruct((BATCH, EMBED), jnp.float32),
        jax.ShapeDtypeStruct((BATCH, EMBED), jnp.float32),
        jax.ShapeDtypeStruct((BATCH, EMBED), jnp.float32),
    ],
    scratch_types=[
        pltpu.VMEM((NCHUNK, CHUNK), jnp.int32),
        pltpu.VMEM((NCHUNK, CHUNK), jnp.int32),
        pltpu.VMEM((NCHUNK, CHUNK), jnp.int32),
        pltpu.VMEM((B_PER_W, EMBED), jnp.float32),
        pltpu.VMEM((B_PER_W, EMBED), jnp.float32),
        pltpu.VMEM((B_PER_W, EMBED), jnp.float32),
        pltpu.SemaphoreType.DMA,
        pltpu.SemaphoreType.DMA,
    ],
)
def _gather3(users_hbm, pos_hbm, neg_hbm, uemb_hbm, iemb_hbm,
             out_u, out_p, out_n,
             idx_u, idx_p, idx_n, rows_u, rows_p, rows_n, sem_g, sem_s):
    wid = lax.axis_index("s") * NC + lax.axis_index("c")
    base = wid * B_PER_W

    pltpu.sync_copy(users_hbm.at[wid], idx_u)
    pltpu.sync_copy(pos_hbm.at[wid], idx_p)
    pltpu.sync_copy(neg_hbm.at[wid], idx_n)

    gathers = []
    for j in range(NCHUNK):
        sl = pl.ds(j * CHUNK, CHUNK)
        gathers.append(pltpu.async_copy(uemb_hbm.at[idx_u.at[j]], rows_u.at[sl], sem_g))
        gathers.append(pltpu.async_copy(iemb_hbm.at[idx_p.at[j]], rows_p.at[sl], sem_g))
        gathers.append(pltpu.async_copy(iemb_hbm.at[idx_n.at[j]], rows_n.at[sl], sem_g))
    for g in gathers:
        g.wait()

    out_sl = pl.ds(base, B_PER_W)
    stores = [
        pltpu.async_copy(rows_u, out_u.at[out_sl], sem_s),
        pltpu.async_copy(rows_p, out_p.at[out_sl], sem_s),
        pltpu.async_copy(rows_n, out_n.at[out_sl], sem_s),
    ]
    for s in stores:
        s.wait()


def kernel(users, pos_items, neg_items, _, user_emb, item_emb):
    u = users.astype(jnp.int32).reshape(NW, NCHUNK, CHUNK)
    p = pos_items.astype(jnp.int32).reshape(NW, NCHUNK, CHUNK)
    n = neg_items.astype(jnp.int32).reshape(NW, NCHUNK, CHUNK)
    out_u, out_p, out_n = _gather3(u, p, n, user_emb, item_emb)
    return out_u, out_p, out_n, _


# speedup vs baseline: 13.2885x; 1.4951x over previous
import functools

import jax
import jax.numpy as jnp
from jax import lax
from jax.experimental import pallas as pl
from jax.experimental.pallas import tpu as pltpu
from jax.experimental.pallas import tpu_sc as plsc

EMBED = 64
BATCH = 16384
TROWS = 1000000

NC = 2
NS = 16
NW = NC * NS
B_PER_W = BATCH // NW
VL = 16
WAVE = 8
NWAVE = B_PER_W // WAVE

_mesh = plsc.VectorSubcoreMesh(core_axis_name="c", subcore_axis_name="s")


@functools.partial(
    pl.kernel,
    mesh=_mesh,
    compiler_params=pltpu.CompilerParams(needs_layout_passes=False),
    out_type=[
        jax.ShapeDtypeStruct((EMBED, BATCH), jnp.float32),
        jax.ShapeDtypeStruct((EMBED, BATCH), jnp.float32),
        jax.ShapeDtypeStruct((EMBED, BATCH), jnp.float32),
    ],
    scratch_types=[
        pltpu.VMEM((B_PER_W,), jnp.int32),
        pltpu.VMEM((WAVE, EMBED, 128), jnp.float32),
        pltpu.VMEM((EMBED, B_PER_W), jnp.float32),
        pltpu.SemaphoreType.DMA,
        pltpu.SemaphoreType.DMA,
    ],
)
def _gather3(users_hbm, pos_hbm, neg_hbm, uemb_hbm, iemb_hbm,
             out_u, out_p, out_n,
             idx_v, grp, out_st, sem_g, sem_s):
    wid = lax.axis_index("s") * NC + lax.axis_index("c")
    base = wid * B_PER_W
    out_sl = pl.ds(base, B_PER_W)
    lanes = lax.iota(jnp.int32, VL)

    def one_table(idx_hbm, tab_hbm, out_hbm):
        pltpu.sync_copy(idx_hbm.at[pl.ds(base, B_PER_W)], idx_v)

        def wave(w, _):
            i0 = w * VL
            evv = idx_v[pl.ds(i0, VL)]
            sv = lax.bitwise_and(evv, 127)
            for h in range(VL // WAVE):
                copies = []
                for j in range(WAVE):
                    e = evv[h * WAVE + j]
                    off = pl.multiple_of(
                        lax.mul(lax.shift_right_logical(e, 7), 128), 128)
                    copies.append(pltpu.async_copy(
                        tab_hbm.at[:, pl.ds(off, 128)], grp.at[j], sem_g))
                for c in copies:
                    c.wait()
                for j in range(WAVE):
                    lane_vec = lax.broadcast(sv[h * WAVE + j], (VL,))
                    col_vec = lax.broadcast(i0 + h * WAVE + j, (VL,))
                    for m in range(EMBED // VL):
                        rows = lax.add(lax.broadcast(m * VL, (VL,)), lanes)
                        v = plsc.load_gather(grp.at[j], [rows, lane_vec])
                        plsc.store_scatter(out_st, [rows, col_vec], v)

        lax.fori_loop(0, B_PER_W // VL, wave, None)

        pltpu.async_copy(out_st, out_hbm.at[:, out_sl], sem_s).wait()

    one_table(users_hbm, uemb_hbm, out_u)
    one_table(pos_hbm, iemb_hbm, out_p)
    one_table(neg_hbm, iemb_hbm, out_n)


def kernel(users, pos_items, neg_items, _, user_emb, item_emb):
    u = users.astype(jnp.int32)
    p = pos_items.astype(jnp.int32)
    n = neg_items.astype(jnp.int32)
    out_u, out_p, out_n = _gather3(u, p, n, user_emb.T, item_emb.T)
    return out_u.T, out_p.T, out_n.T, _


# speedup vs baseline: 16.0988x; 1.2115x over previous
import functools

import jax
import jax.numpy as jnp
from jax import lax
from jax.experimental import pallas as pl
from jax.experimental.pallas import tpu as pltpu
from jax.experimental.pallas import tpu_sc as plsc

EMBED = 64
BATCH = 16384
TROWS = 1000000

NC = 2
NS = 16
NW = NC * NS
B_PER_W = BATCH // NW
VL = 16
WAVE = 4
NWAVE = B_PER_W // WAVE

_mesh = plsc.VectorSubcoreMesh(core_axis_name="c", subcore_axis_name="s")


@functools.partial(
    pl.kernel,
    mesh=_mesh,
    compiler_params=pltpu.CompilerParams(needs_layout_passes=False),
    out_type=[
        jax.ShapeDtypeStruct((EMBED, BATCH), jnp.float32),
        jax.ShapeDtypeStruct((EMBED, BATCH), jnp.float32),
        jax.ShapeDtypeStruct((EMBED, BATCH), jnp.float32),
    ],
    scratch_types=[
        pltpu.VMEM((B_PER_W,), jnp.int32),
        pltpu.VMEM((2, WAVE, EMBED, 128), jnp.float32),
        pltpu.VMEM((EMBED, B_PER_W), jnp.float32),
        pltpu.SemaphoreType.DMA,
        pltpu.SemaphoreType.DMA,
    ],
)
def _gather3(users_hbm, pos_hbm, neg_hbm, uemb_hbm, iemb_hbm,
             out_u, out_p, out_n,
             idx_v, grp, out_st, sem_g, sem_s):
    wid = lax.axis_index("s") * NC + lax.axis_index("c")
    base = wid * B_PER_W
    out_sl = pl.ds(base, B_PER_W)
    lanes = lax.iota(jnp.int32, VL)

    def one_table(idx_hbm, tab_hbm, out_hbm):
        pltpu.sync_copy(idx_hbm.at[pl.ds(base, B_PER_W)], idx_v)

        def wave(w, _):
            i0 = w * VL
            evv = idx_v[pl.ds(i0, VL)]
            sv = lax.bitwise_and(evv, 127)
            nb = VL // WAVE

            def fire(h):
                copies = []
                for j in range(WAVE):
                    e = evv[h * WAVE + j]
                    off = pl.multiple_of(
                        lax.mul(lax.shift_right_logical(e, 7), 128), 128)
                    copies.append(pltpu.async_copy(
                        tab_hbm.at[:, pl.ds(off, 128)],
                        grp.at[h % 2, j], sem_g))
                return copies

            copies = fire(0)
            for h in range(nb):
                nxt = fire(h + 1) if h + 1 < nb else []
                for c in copies:
                    c.wait()
                copies = nxt
                for j in range(WAVE):
                    lane_vec = lax.broadcast(sv[h * WAVE + j], (VL,))
                    col_vec = lax.broadcast(i0 + h * WAVE + j, (VL,))
                    for m in range(EMBED // VL):
                        rows = lax.add(lax.broadcast(m * VL, (VL,)), lanes)
                        v = plsc.load_gather(grp.at[h % 2, j],
                                             [rows, lane_vec])
                        plsc.store_scatter(out_st, [rows, col_vec], v)

        lax.fori_loop(0, B_PER_W // VL, wave, None)

        pltpu.async_copy(out_st, out_hbm.at[:, out_sl], sem_s).wait()

    one_table(users_hbm, uemb_hbm, out_u)
    one_table(pos_hbm, iemb_hbm, out_p)
    one_table(neg_hbm, iemb_hbm, out_n)


def kernel(users, pos_items, neg_items, _, user_emb, item_emb):
    u = users.astype(jnp.int32)
    p = pos_items.astype(jnp.int32)
    n = neg_items.astype(jnp.int32)
    out_u, out_p, out_n = _gather3(u, p, n, user_emb.T, item_emb.T)
    return out_u.T, out_p.T, out_n.T, _


# speedup vs baseline: 17.1352x; 1.0644x over previous
import functools

import jax
import jax.numpy as jnp
from jax import lax
from jax.experimental import pallas as pl
from jax.experimental.pallas import tpu as pltpu
from jax.experimental.pallas import tpu_sc as plsc

EMBED = 64
BATCH = 16384
TROWS = 1000000

NC = 2
NS = 16
NW = NC * NS
B_PER_W = BATCH // NW
VL = 16
WAVE = 4
NWAVE = B_PER_W // WAVE

_mesh = plsc.VectorSubcoreMesh(core_axis_name="c", subcore_axis_name="s")


@functools.partial(
    pl.kernel,
    mesh=_mesh,
    compiler_params=pltpu.CompilerParams(needs_layout_passes=False),
    out_type=[
        jax.ShapeDtypeStruct((EMBED, BATCH), jnp.float32),
        jax.ShapeDtypeStruct((EMBED, BATCH), jnp.float32),
        jax.ShapeDtypeStruct((EMBED, BATCH), jnp.float32),
    ],
    scratch_types=[
        pltpu.VMEM((B_PER_W,), jnp.int32),
        pltpu.VMEM((2, WAVE, EMBED, 128), jnp.float32),
        pltpu.VMEM((EMBED, B_PER_W), jnp.float32),
        pltpu.SemaphoreType.DMA,
        pltpu.SemaphoreType.DMA,
    ],
)
def _gather3(users_hbm, pos_hbm, neg_hbm, uemb_hbm, iemb_hbm,
             out_u, out_p, out_n,
             idx_v, grp, out_st, sem_g, sem_s):
    wid = lax.axis_index("s") * NC + lax.axis_index("c")
    base = wid * B_PER_W
    out_sl = pl.ds(base, B_PER_W)
    lanes = lax.iota(jnp.int32, VL)

    def one_table(idx_hbm, tab_hbm, out_hbm):
        pltpu.sync_copy(idx_hbm.at[pl.ds(base, B_PER_W)], idx_v)

        def wave(w, _):
            i0 = w * 2 * VL
            evs = [idx_v[pl.ds(i0, VL)], idx_v[pl.ds(i0 + VL, VL)]]
            svs = [lax.bitwise_and(ev, 127) for ev in evs]
            nb = 2 * VL // WAVE

            def fire(h):
                ev = evs[(h * WAVE) // VL]
                copies = []
                for j in range(WAVE):
                    e = ev[(h * WAVE) % VL + j]
                    off = pl.multiple_of(
                        lax.mul(lax.shift_right_logical(e, 7), 128), 128)
                    copies.append(pltpu.async_copy(
                        tab_hbm.at[:, pl.ds(off, 128)],
                        grp.at[h % 2, j], sem_g))
                return copies

            copies = fire(0)
            for h in range(nb):
                nxt = fire(h + 1) if h + 1 < nb else []
                for c in copies:
                    c.wait()
                copies = nxt
                sv = svs[(h * WAVE) // VL]
                for j in range(WAVE):
                    lane_vec = lax.broadcast(sv[(h * WAVE) % VL + j], (VL,))
                    col_vec = lax.broadcast(i0 + h * WAVE + j, (VL,))
                    for m in range(EMBED // VL):
                        rows = lax.add(lax.broadcast(m * VL, (VL,)), lanes)
                        v = plsc.load_gather(grp.at[h % 2, j],
                                             [rows, lane_vec])
                        plsc.store_scatter(out_st, [rows, col_vec], v)

        lax.fori_loop(0, B_PER_W // (2 * VL), wave, None)

        pltpu.async_copy(out_st, out_hbm.at[:, out_sl], sem_s).wait()

    one_table(users_hbm, uemb_hbm, out_u)
    one_table(pos_hbm, iemb_hbm, out_p)
    one_table(neg_hbm, iemb_hbm, out_n)


def kernel(users, pos_items, neg_items, _, user_emb, item_emb):
    u = users.astype(jnp.int32)
    p = pos_items.astype(jnp.int32)
    n = neg_items.astype(jnp.int32)
    out_u, out_p, out_n = _gather3(u, p, n, user_emb.T, item_emb.T)
    return out_u.T, out_p.T, out_n.T, _
